# Initial kernel scaffold; baseline (speedup 1.0000x reference)
import jax
import jax.numpy as jnp
from jax.experimental import pallas as pl


def kernel(x, table):
    raise NotImplementedError("write your pallas kernel here")


# speedup vs baseline: 4.1748x; 4.1748x over previous
import functools

import jax
import jax.numpy as jnp
from jax import lax
from jax.experimental import pallas as pl
from jax.experimental.pallas import tpu as pltpu
from jax.experimental.pallas import tpu_sc as plsc

_NUM_CORES = 2
_NUM_SUBCORES = 16
_NUM_WORKERS = _NUM_CORES * _NUM_SUBCORES


@functools.lru_cache(maxsize=None)
def _make_gather(n, d, chunk):
    per_worker = n // _NUM_WORKERS
    nchunks = per_worker // chunk
    npairs = nchunks // 2
    mesh = plsc.VectorSubcoreMesh(core_axis_name="c", subcore_axis_name="s")

    @functools.partial(
        pl.kernel,
        mesh=mesh,
        compiler_params=pltpu.CompilerParams(use_tc_tiling_on_sc=False),
        out_type=jax.ShapeDtypeStruct((n, d), jnp.float32),
        scratch_types=[
            pltpu.VMEM((chunk,), jnp.int32),
            pltpu.VMEM((chunk,), jnp.int32),
            pltpu.VMEM((chunk, d), jnp.float32),
            pltpu.VMEM((chunk, d), jnp.float32),
            pltpu.SemaphoreType.DMA,
            pltpu.SemaphoreType.DMA,
        ],
    )
    def gather_kernel(table_hbm, idx_hbm, out_hbm,
                      idx0, idx1, rows0, rows1, g0, g1):
        wid = lax.axis_index("s") * _NUM_CORES + lax.axis_index("c")
        base = wid * per_worker

        def load_idx(buf, c):
            pltpu.sync_copy(idx_hbm.at[pl.ds(base + c * chunk, chunk)], buf)

        load_idx(idx0, 0)
        pltpu.async_copy(table_hbm.at[idx0], rows0, g0)

        def pair(j, carry):
            c0 = 2 * j
            c1 = c0 + 1
            load_idx(idx1, c1)
            pltpu.async_copy(table_hbm.at[idx1], rows1, g1)
            pltpu.make_async_copy(table_hbm.at[idx0], rows0, g0).wait()
            pltpu.sync_copy(rows0, out_hbm.at[pl.ds(base + c0 * chunk, chunk)])
            c2 = jnp.minimum(c1 + 1, nchunks - 1)
            load_idx(idx0, c2)
            pltpu.async_copy(table_hbm.at[idx0], rows0, g0)
            pltpu.make_async_copy(table_hbm.at[idx1], rows1, g1).wait()
            pltpu.sync_copy(rows1, out_hbm.at[pl.ds(base + c1 * chunk, chunk)])
            return carry

        lax.fori_loop(0, npairs, pair, 0)
        pltpu.make_async_copy(table_hbm.at[idx0], rows0, g0).wait()

    return gather_kernel


@jax.jit
def kernel(x, table):
    n = x.size
    d = table.shape[1]
    flat = x.reshape((n,)).astype(jnp.int32)
    out = _make_gather(n, d, 512)(table, flat)
    return out.reshape(x.shape + (d,))


# speedup vs baseline: 4.2699x; 1.0228x over previous
import functools

import jax
import jax.numpy as jnp
from jax import lax
from jax.experimental import pallas as pl
from jax.experimental.pallas import tpu as pltpu
from jax.experimental.pallas import tpu_sc as plsc

_NUM_CORES = 2
_NUM_SUBCORES = 16
_NUM_WORKERS = _NUM_CORES * _NUM_SUBCORES
_NBUF = 4


@functools.lru_cache(maxsize=None)
def _make_gather(n, d, chunk):
    per_worker = n // _NUM_WORKERS
    nchunks = per_worker // chunk
    npj = nchunks // _NBUF
    assert nchunks % _NBUF == 0 and npj >= 2
    mesh = plsc.VectorSubcoreMesh(core_axis_name="c", subcore_axis_name="s")

    @functools.partial(
        pl.kernel,
        mesh=mesh,
        compiler_params=pltpu.CompilerParams(use_tc_tiling_on_sc=False),
        out_type=jax.ShapeDtypeStruct((n, d), jnp.float32),
        scratch_types=[
            pltpu.VMEM((_NBUF, chunk), jnp.int32),
            pltpu.VMEM((_NBUF, chunk, d), jnp.float32),
        ] + [pltpu.SemaphoreType.DMA] * (2 * _NBUF),
    )
    def gather_kernel(table_hbm, idx_hbm, out_hbm, idxb, rowsb,
                      g0, g1, g2, g3, w0, w1, w2, w3):
        gs = (g0, g1, g2, g3)
        ws = (w0, w1, w2, w3)
        wid = lax.axis_index("s") * _NUM_CORES + lax.axis_index("c")
        base = wid * per_worker

        def out_at(c):
            return out_hbm.at[pl.ds(base + c * chunk, chunk)]

        def fire_gather(c, b):
            pltpu.sync_copy(idx_hbm.at[pl.ds(base + c * chunk, chunk)],
                            idxb.at[b])
            pltpu.async_copy(table_hbm.at[idxb.at[b]], rowsb.at[b], gs[b])

        def wait_gather(b):
            pltpu.make_async_copy(table_hbm.at[idxb.at[b]], rowsb.at[b],
                                  gs[b]).wait()

        def fire_write(c, b):
            pltpu.async_copy(rowsb.at[b], out_at(c), ws[b])

        def wait_write(c, b):
            pltpu.make_async_copy(rowsb.at[b], out_at(c), ws[b]).wait()

        fire_gather(0, 0)
        fire_gather(1, 1)

        def slot(c, b):
            bn = (b + 2) % _NBUF
            wait_write(c - 2, bn)
            fire_gather(c + 2, bn)
            wait_gather(b)
            fire_write(c, b)

        fire_gather(2, 2)
        wait_gather(0)
        fire_write(0, 0)
        fire_gather(3, 3)
        wait_gather(1)
        fire_write(1, 1)
        slot(2, 2)
        slot(3, 3)

        def body(j, carry):
            c = j * _NBUF
            slot(c, 0)
            slot(c + 1, 1)
            slot(c + 2, 2)
            slot(c + 3, 3)
            return carry

        lax.fori_loop(1, npj - 1, body, 0)

        c = nchunks - 4
        slot(c, 0)
        slot(c + 1, 1)
        wait_gather(2)
        fire_write(c + 2, 2)
        wait_gather(3)
        fire_write(c + 3, 3)
        for b in range(_NBUF):
            wait_write(nchunks - 4 + b, b)

    return gather_kernel


@jax.jit
def kernel(x, table):
    n = x.size
    d = table.shape[1]
    flat = x.reshape((n,)).astype(jnp.int32)
    out = _make_gather(n, d, 400)(table, flat)
    return out.reshape(x.shape + (d,))
